# Initial kernel scaffold; baseline (speedup 1.0000x reference)
import jax
import jax.numpy as jnp
from jax.experimental import pallas as pl


def kernel(features, coords):
    raise NotImplementedError("write your pallas kernel here")


# speedup vs baseline: 2.4160x; 2.4160x over previous
import functools

import jax
import jax.numpy as jnp
from jax import lax
from jax.experimental import pallas as pl
from jax.experimental.pallas import tpu as pltpu
from jax.experimental.pallas import tpu_sc as plsc

B, F, N = 8, 16, 100000
R = 32
NV = R * R * R
NP = 100352
TRASH = NV
NACC = NV + 256
EPS = 1e-6

NSUB = 16
NCORE = 2
BPC = B // NCORE
PPS = NP // NSUB
KROWS = 128
NCHUNK = PPS // KROWS
G_OUT = 7
G_IN = NCHUNK // G_OUT
RB = G_IN * KROWS
ZROWS = NACC // NSUB
OROWS = NV // NSUB


def _coords_body(c_ref, nc_ref, idx_ref):
    c = c_ref[0]
    mean = jnp.sum(c, axis=1, keepdims=True) * (1.0 / N)
    cent = c - mean
    sq = jnp.sum(cent * cent, axis=0, keepdims=True)
    pos = lax.broadcasted_iota(jnp.int32, (1, NP), 1)
    valid = pos < N
    mx = jnp.sqrt(jnp.max(jnp.where(valid, sq, -1.0)))
    scale = 1.0 / (2.0 * mx + EPS)
    ncd = cent * scale + 0.5
    scl = jnp.clip(ncd * R, 0.0, float(R - 1))
    nc_ref[0] = scl[:, :N]
    vox = jnp.round(scl).astype(jnp.int32)
    fi = vox[0:1] * (R * R) + vox[1:2] * R + vox[2:3]
    idx_ref[0] = jnp.where(valid, fi, TRASH)


def _coords_tc(coords_p):
    return pl.pallas_call(
        _coords_body,
        grid=(B,),
        in_specs=[pl.BlockSpec((1, 3, NP), lambda b: (b, 0, 0))],
        out_specs=[
            pl.BlockSpec((1, 3, N), lambda b: (b, 0, 0)),
            pl.BlockSpec((1, 1, NP), lambda b: (b, 0, 0)),
        ],
        out_shape=[
            jax.ShapeDtypeStruct((B, 3, N), jnp.float32),
            jax.ShapeDtypeStruct((B, 1, NP), jnp.int32),
        ],
    )(coords_p)


_WT = 1024
_NT = NP // _WT


def _tr_body(f_ref, o_ref):
    o_ref[0] = f_ref[0].T


def _transpose_tc(features):
    return pl.pallas_call(
        _tr_body,
        grid=(B, _NT),
        in_specs=[pl.BlockSpec((1, F, _WT), lambda b, i: (b, 0, i))],
        out_specs=pl.BlockSpec((1, _WT, F), lambda b, i: (b, i, 0)),
        out_shape=jax.ShapeDtypeStruct((B, NP, F), jnp.float32),
    )(features)


def _sc_body(ft_hbm, idx_hbm, sums_hbm, cnt_hbm,
             rows_v, idx_v, zeros_v, czero_v, ones_v, sums_sh, cnt_sh):
    c = lax.axis_index("c")
    s = lax.axis_index("s")

    zvec = jnp.zeros((16,), jnp.float32)
    ovec = jnp.ones((16,), jnp.float32)

    @pl.loop(0, ZROWS // 2)
    def _(i):
        zeros_v[i, :] = zvec

    @pl.loop(0, ZROWS // 16)
    def _(i):
        czero_v[pl.ds(i * 16, 16)] = zvec

    @pl.loop(0, KROWS // 16)
    def _(i):
        ones_v[pl.ds(i * 16, 16)] = ovec

    for b in range(BPC):
        batch = c * BPC + b

        base = s * ZROWS
        pltpu.sync_copy(zeros_v, sums_sh.at[pl.ds(base, ZROWS // 2), :])
        pltpu.sync_copy(zeros_v,
                        sums_sh.at[pl.ds(base + ZROWS // 2, ZROWS // 2), :])
        pltpu.sync_copy(czero_v, cnt_sh.at[pl.ds(base, ZROWS)])
        plsc.subcore_barrier()

        pltpu.sync_copy(idx_hbm.at[batch, s], idx_v)

        for g in range(G_OUT):
            pltpu.sync_copy(
                ft_hbm.at[batch, pl.ds(s * PPS + g * RB, RB), :], rows_v)
            for j in range(G_IN):
                jj = g * G_IN + j
                pltpu.sync_copy(rows_v.at[pl.ds(j * KROWS, KROWS), :],
                                sums_sh.at[idx_v.at[jj]], add=True)
                pltpu.sync_copy(ones_v, cnt_sh.at[idx_v.at[jj]], add=True)
        plsc.subcore_barrier()

        obase = s * OROWS
        pltpu.sync_copy(sums_sh.at[pl.ds(obase, OROWS), :],
                        sums_hbm.at[batch, pl.ds(obase, OROWS), :])
        pltpu.sync_copy(cnt_sh.at[pl.ds(obase, OROWS)],
                        cnt_hbm.at[batch, pl.ds(obase, OROWS)])
        plsc.subcore_barrier()


def _sc_scatter(ft, idx3):
    mesh = plsc.VectorSubcoreMesh(core_axis_name="c", subcore_axis_name="s")
    kern = pl.kernel(
        _sc_body,
        mesh=mesh,
        compiler_params=pltpu.CompilerParams(use_tc_tiling_on_sc=False),
        out_type=[
            jax.ShapeDtypeStruct((B, NV, F), jnp.float32),
            jax.ShapeDtypeStruct((B, NV), jnp.float32),
        ],
        scratch_types=[
            pltpu.VMEM((RB, F), jnp.float32),
            pltpu.VMEM((NCHUNK, KROWS), jnp.int32),
            pltpu.VMEM((ZROWS // 2, F), jnp.float32),
            pltpu.VMEM((ZROWS,), jnp.float32),
            pltpu.VMEM((KROWS,), jnp.float32),
            pltpu.VMEM_SHARED((NACC, F), jnp.float32),
            pltpu.VMEM_SHARED((NACC,), jnp.float32),
        ],
    )
    return kern(ft, idx3)


_WV = 2048


def _fin_body(s_ref, c_ref, o_ref):
    sums = s_ref[0]
    cnt = jnp.maximum(c_ref[0], 1.0).reshape(1, _WV)
    o_ref[0] = sums.T * (1.0 / cnt)


def _finalize_tc(sums, cnt):
    cnt3 = cnt.reshape(B, NV // 128, 128)
    return pl.pallas_call(
        _fin_body,
        grid=(B, NV // _WV),
        in_specs=[
            pl.BlockSpec((1, _WV, F), lambda b, i: (b, i, 0)),
            pl.BlockSpec((1, _WV // 128, 128), lambda b, i: (b, i, 0)),
        ],
        out_specs=pl.BlockSpec((1, F, _WV), lambda b, i: (b, 0, i)),
        out_shape=jax.ShapeDtypeStruct((B, F, NV), jnp.float32),
    )(sums, cnt3)


def kernel(features, coords):
    coords_p = jnp.pad(coords, ((0, 0), (0, 0), (0, NP - N)))
    nc, idx = _coords_tc(coords_p)
    ft = _transpose_tc(features)
    idx4 = idx.reshape(B, NSUB, NCHUNK, KROWS)
    sums, cnt = _sc_scatter(ft, idx4)
    out = _finalize_tc(sums, cnt)
    return out.reshape(B, F, R, R, R), nc
